# baseline (speedup 1.0000x reference)
import functools

import jax
import jax.numpy as jnp
from jax import lax
from jax.experimental import pallas as pl
from jax.experimental.pallas import tpu as pltpu
from jax.experimental.pallas import tpu_sc as plsc

N = 50000
E = 800000
IN = 115
H1 = 32
H2 = 16
NB = 50
HC2 = 64

NC = 2
NS = 16
NW = NC * NS
C = 128
EW = 25600
EP = NW * EW
NCHUNK = EW // C
NP = 51200
RT = NP // NS
ZC = 128


def _leaky(v):
    return jnp.maximum(v, 0.2 * v)


def _edge_sweep(H):
    mesh = plsc.VectorSubcoreMesh(core_axis_name="c", subcore_axis_name="s")

    NPH = 4
    dma = pltpu.SemaphoreType.DMA

    @functools.partial(
        pl.kernel,
        out_type=[
            jax.ShapeDtypeStruct((NC, NP, H), jnp.float32),
            jax.ShapeDtypeStruct((NC * NP,), jnp.float32),
        ],
        mesh=mesh,
        compiler_params=pltpu.CompilerParams(use_tc_tiling_on_sc=False),
        scratch_types=(
            [pltpu.VMEM_SHARED((NP, H), jnp.float32),
             pltpu.VMEM_SHARED((NP,), jnp.float32)]
            + [pltpu.VMEM((C,), jnp.int32) for _ in range(2 * NPH)]
            + [pltpu.VMEM((C,), jnp.float32) for _ in range(3 * NPH)]
            + [pltpu.VMEM((C, H), jnp.float32) for _ in range(NPH)]
            + [dma for _ in range(7 * NPH)]
        ),
    )
    def sweep(src_r, dst_r, as_r, ad_r, xp_r, acc_o, den_o,
              acc_sh, den_sh, *scr):
        srcv = scr[0:NPH]
        dstv = scr[NPH:2 * NPH]
        asv = scr[2 * NPH:3 * NPH]
        adv = scr[3 * NPH:4 * NPH]
        wv = scr[4 * NPH:5 * NPH]
        rowsv = scr[5 * NPH:6 * NPH]
        sems = scr[6 * NPH:]
        s_src = sems[0:NPH]
        s_dst = sems[NPH:2 * NPH]
        s_as = sems[2 * NPH:3 * NPH]
        s_ad = sems[3 * NPH:4 * NPH]
        s_rows = sems[4 * NPH:5 * NPH]
        s_sa = sems[5 * NPH:6 * NPH]
        s_sd = sems[6 * NPH:7 * NPH]

        cid = lax.axis_index("c")
        sid = lax.axis_index("s")
        wid = sid * NC + cid
        ebase = wid * EW
        r0 = sid * RT

        z16 = jnp.zeros((16,), jnp.float32)
        def zfill(i, _):
            for j in range(H // 16):
                scr[5 * NPH][i, pl.ds(j * 16, 16)] = z16
            return 0
        lax.fori_loop(0, ZC, zfill, 0)
        for j in range(ZC // 16):
            scr[4 * NPH][pl.ds(j * 16, 16)] = z16
        def zfire(k, _):
            ro = r0 + k * ZC
            pltpu.async_copy(scr[5 * NPH], acc_sh.at[pl.ds(ro, ZC)],
                             scr[6 * NPH + 5 * NPH])
            pltpu.async_copy(scr[4 * NPH], den_sh.at[pl.ds(ro, ZC)],
                             scr[6 * NPH + 6 * NPH])
            return 0
        lax.fori_loop(0, RT // ZC, zfire, 0)
        def zwait(k, _):
            ro = r0 + k * ZC
            pltpu.make_async_copy(scr[5 * NPH], acc_sh.at[pl.ds(ro, ZC)],
                                  scr[6 * NPH + 5 * NPH]).wait()
            pltpu.make_async_copy(scr[4 * NPH], den_sh.at[pl.ds(ro, ZC)],
                                  scr[6 * NPH + 6 * NPH]).wait()
            return 0
        lax.fori_loop(0, RT // ZC, zwait, 0)
        plsc.subcore_barrier()

        def fire_idx(i, p):
            o = ebase + i * C
            pltpu.async_copy(src_r.at[pl.ds(o, C)], srcv[p], s_src[p])
            pltpu.async_copy(dst_r.at[pl.ds(o, C)], dstv[p], s_dst[p])

        def wait_idx(p):
            pltpu.make_async_copy(src_r.at[pl.ds(ebase, C)], srcv[p], s_src[p]).wait()
            pltpu.make_async_copy(dst_r.at[pl.ds(ebase, C)], dstv[p], s_dst[p]).wait()

        def fire_gathers(p):
            pltpu.async_copy(as_r.at[srcv[p]], asv[p], s_as[p])
            pltpu.async_copy(ad_r.at[dstv[p]], adv[p], s_ad[p])
            pltpu.async_copy(xp_r.at[srcv[p]], rowsv[p], s_rows[p])

        def wait_gathers(p):
            pltpu.make_async_copy(as_r.at[srcv[p]], asv[p], s_as[p]).wait()
            pltpu.make_async_copy(ad_r.at[dstv[p]], adv[p], s_ad[p]).wait()
            pltpu.make_async_copy(xp_r.at[srcv[p]], rowsv[p], s_rows[p]).wait()

        def fire_scatters(p):
            pltpu.async_copy(rowsv[p], acc_sh.at[dstv[p]], s_sa[p], add=True)
            pltpu.async_copy(wv[p], den_sh.at[dstv[p]], s_sd[p], add=True)

        def wait_scatters(p):
            pltpu.make_async_copy(rowsv[p], acc_sh.at[dstv[p]], s_sa[p]).wait()
            pltpu.make_async_copy(wv[p], den_sh.at[dstv[p]], s_sd[p]).wait()

        def compute(p):
            for k in range(C // 16):
                sl = pl.ds(k * 16, 16)
                e = _leaky(asv[p][sl] + adv[p][sl])
                wv[p][sl] = jnp.exp(e)
            def sgroup(g, _):
                w16 = wv[p][pl.ds(g * 16, 16)]
                for l in range(16):
                    i = g * 16 + l
                    w = w16[l]
                    for j in range(H // 16):
                        sl = pl.ds(j * 16, 16)
                        rowsv[p][i, sl] = rowsv[p][i, sl] * w
                return 0
            lax.fori_loop(0, C // 16, sgroup, 0)

        fire_idx(0, 0)
        fire_idx(1, 1)
        wait_idx(0)
        fire_gathers(0)

        def step(j, _):
            for l in range(NPH):
                i = j * NPH + l
                p1 = (l + 1) % NPH
                p2 = (l + 2) % NPH

                @pl.when(i + 2 < NCHUNK)
                def _():
                    @pl.when(i >= 2)
                    def _():
                        wait_scatters(p2)
                    fire_idx(i + 2, p2)

                @pl.when(i + 1 < NCHUNK)
                def _():
                    wait_idx(p1)
                    fire_gathers(p1)

                wait_gathers(l)
                compute(l)
                fire_scatters(l)
            return 0
        lax.fori_loop(0, NCHUNK // NPH, step, 0)
        for p in range(NPH):
            wait_scatters(p)
        plsc.subcore_barrier()

        pltpu.sync_copy(acc_sh.at[pl.ds(r0, RT)], acc_o.at[cid, pl.ds(r0, RT)])
        pltpu.sync_copy(den_sh.at[pl.ds(r0, RT)],
                        den_o.at[pl.ds(cid * NP + r0, RT)])

    return sweep


_BLK = 5000
_GRID = N // _BLK


def _proj1_body(x_ref, w_ref, s_ref, d_ref, xp_ref, as_ref, ad_ref):
    xp = jnp.dot(x_ref[...], w_ref[...], preferred_element_type=jnp.float32)
    xp_ref[...] = xp
    as_ref[...] = jnp.dot(xp, s_ref[...], preferred_element_type=jnp.float32)
    ad_ref[...] = jnp.dot(xp, d_ref[...], preferred_element_type=jnp.float32)


def _proj1(x, W1t, att_s, att_d):
    return pl.pallas_call(
        _proj1_body,
        grid=(_GRID,),
        in_specs=[
            pl.BlockSpec((_BLK, IN), lambda i: (i, 0)),
            pl.BlockSpec((IN, H1), lambda i: (0, 0)),
            pl.BlockSpec((H1, 1), lambda i: (0, 0)),
            pl.BlockSpec((H1, 1), lambda i: (0, 0)),
        ],
        out_specs=[
            pl.BlockSpec((_BLK, H1), lambda i: (i, 0)),
            pl.BlockSpec((_BLK, 1), lambda i: (i, 0)),
            pl.BlockSpec((_BLK, 1), lambda i: (i, 0)),
        ],
        out_shape=[
            jax.ShapeDtypeStruct((N, H1), jnp.float32),
            jax.ShapeDtypeStruct((NP, 1), jnp.float32),
            jax.ShapeDtypeStruct((NP, 1), jnp.float32),
        ],
    )(x, W1t, att_s, att_d)


def _combine2_body(acc_ref, den_ref, b_ref, w_ref, s_ref, d_ref,
                   xp_ref, as_ref, ad_ref):
    a = acc_ref[0] + acc_ref[1]
    d = den_ref[0] + den_ref[1]
    h = jnp.maximum(a / (d + 1e-16) + b_ref[...], 0.0)
    xp = jnp.dot(h, w_ref[...], preferred_element_type=jnp.float32)
    xp_ref[...] = xp
    as_ref[...] = jnp.dot(xp, s_ref[...], preferred_element_type=jnp.float32)
    ad_ref[...] = jnp.dot(xp, d_ref[...], preferred_element_type=jnp.float32)


def _combine_proj2(acc, den, b1, W2t, att_s, att_d):
    return pl.pallas_call(
        _combine2_body,
        grid=(_GRID,),
        in_specs=[
            pl.BlockSpec((NC, _BLK, H1), lambda i: (0, i, 0)),
            pl.BlockSpec((NC, _BLK, 1), lambda i: (0, i, 0)),
            pl.BlockSpec((1, H1), lambda i: (0, 0)),
            pl.BlockSpec((H1, H2), lambda i: (0, 0)),
            pl.BlockSpec((H2, 1), lambda i: (0, 0)),
            pl.BlockSpec((H2, 1), lambda i: (0, 0)),
        ],
        out_specs=[
            pl.BlockSpec((_BLK, H2), lambda i: (i, 0)),
            pl.BlockSpec((_BLK, 1), lambda i: (i, 0)),
            pl.BlockSpec((_BLK, 1), lambda i: (i, 0)),
        ],
        out_shape=[
            jax.ShapeDtypeStruct((N, H2), jnp.float32),
            jax.ShapeDtypeStruct((NP, 1), jnp.float32),
            jax.ShapeDtypeStruct((NP, 1), jnp.float32),
        ],
    )(acc, den, b1, W2t, att_s, att_d)


def _combine3_body(acc_ref, den_ref, b_ref, h_ref):
    a = acc_ref[0] + acc_ref[1]
    d = den_ref[0] + den_ref[1]
    h_ref[...] = jnp.maximum(a / (d + 1e-16) + b_ref[...], 0.0)


def _combine3(acc, den, b2):
    return pl.pallas_call(
        _combine3_body,
        grid=(_GRID,),
        in_specs=[
            pl.BlockSpec((NC, _BLK, H2), lambda i: (0, i, 0)),
            pl.BlockSpec((NC, _BLK, 1), lambda i: (0, i, 0)),
            pl.BlockSpec((1, H2), lambda i: (0, 0)),
        ],
        out_specs=pl.BlockSpec((_BLK, H2), lambda i: (i, 0)),
        out_shape=jax.ShapeDtypeStruct((N, H2), jnp.float32),
    )(acc, den, b2)


def _head_body(h_ref, w1_ref, b1_ref, w3_ref, b3_ref, ba_ref, bb_ref, o_ref):
    z = lax.dot_general(h_ref[...], w1_ref[...],
                        dimension_numbers=(((1,), (1,)), ((), ())),
                        preferred_element_type=jnp.float32)
    z = jnp.maximum(z + b1_ref[...], 0.0)
    o = jnp.dot(z, w3_ref[...], preferred_element_type=jnp.float32) + b3_ref[...]
    nbt = 1.0 + jnp.sum((ba_ref[...] != bb_ref[...]).astype(jnp.float32))
    o_ref[...] = o * (nbt / NB)


def _head(h2f, lin1_Wt, lin1_b, lin3_Wt, lin3_b, ba, bb):
    HC = h2f.shape[1]
    return pl.pallas_call(
        _head_body,
        grid=(1,),
        in_specs=[
            pl.BlockSpec((NB, HC), lambda i: (0, 0)),
            pl.BlockSpec((HC2, HC), lambda i: (0, 0)),
            pl.BlockSpec((1, HC2), lambda i: (0, 0)),
            pl.BlockSpec((HC2, 1), lambda i: (0, 0)),
            pl.BlockSpec((1, 1), lambda i: (0, 0)),
            pl.BlockSpec(ba.shape, lambda i: (0, 0)),
            pl.BlockSpec(bb.shape, lambda i: (0, 0)),
        ],
        out_specs=pl.BlockSpec((NB, 1), lambda i: (0, 0)),
        out_shape=jax.ShapeDtypeStruct((NB, 1), jnp.float32),
    )(h2f, lin1_Wt, lin1_b, lin3_Wt, lin3_b, ba, bb)


def kernel(x, edge_index, edge_weight, batch, device,
           W1, att1_src, att1_dst, b1,
           W2, att2_src, att2_dst, b2,
           lin1_W, lin1_b, lin3_W, lin3_b):
    f32 = jnp.float32
    src = edge_index[0]
    dst = edge_index[1]
    pad = EP - E
    src_p = jnp.concatenate([src, jnp.zeros((pad,), jnp.int32)])
    pad_dst = N + (jnp.arange(pad, dtype=jnp.int32) % (NP - N))
    dst_p = jnp.concatenate([dst, pad_dst])

    xp1, as1, ad1 = _proj1(x, W1.T, att1_src.reshape(H1, 1),
                           att1_dst.reshape(H1, 1))
    acc1, den1 = _edge_sweep(H1)(src_p, dst_p, as1.reshape(NP),
                                 ad1.reshape(NP), xp1)

    xp2, as2, ad2 = _combine_proj2(acc1, den1.reshape(NC, NP, 1),
                                   b1.reshape(1, H1), W2.T,
                                   att2_src.reshape(H2, 1),
                                   att2_dst.reshape(H2, 1))
    acc2, den2 = _edge_sweep(H2)(src_p, dst_p, as2.reshape(NP),
                                 ad2.reshape(NP), xp2)

    h2 = _combine3(acc2, den2.reshape(NC, NP, 1), b2.reshape(1, H2))

    h2f = h2.reshape(NB, (N // NB) * H2)
    ba = batch.reshape(500, 100)
    bb = jnp.concatenate([batch[:1], batch[:-1]]).reshape(500, 100)
    out = _head(h2f, lin1_W, lin1_b.reshape(1, HC2),
                lin3_W.T, lin3_b.reshape(1, 1), ba, bb)
    return out


# speedup vs baseline: 1.7955x; 1.7955x over previous
import functools

import jax
import jax.numpy as jnp
from jax import lax
from jax.experimental import pallas as pl
from jax.experimental.pallas import tpu as pltpu
from jax.experimental.pallas import tpu_sc as plsc

N = 50000
E = 800000
IN = 115
H1 = 32
H2 = 16
NB = 50
HC2 = 64

NC = 2
NS = 16
NW = NC * NS
C = 128
NCHT = E // C
NCH_LO = NCHT // NW
NCH_XT = NCHT % NW
NP = 51200
RT = NP // NS
ZC = 128


def _leaky(v):
    return jnp.maximum(v, 0.2 * v)


def _edge_sweep(H, fold):
    mesh = plsc.VectorSubcoreMesh(core_axis_name="c", subcore_axis_name="s")

    NPH = 4
    AS_COL = H2
    dma = pltpu.SemaphoreType.DMA

    out_type = [jax.ShapeDtypeStruct((NC, NP, H), jnp.float32)]
    shared = [pltpu.VMEM_SHARED((NP, H), jnp.float32)]
    if not fold:
        out_type.append(jax.ShapeDtypeStruct((NC * NP,), jnp.float32))
        shared.append(pltpu.VMEM_SHARED((NP,), jnp.float32))

    @functools.partial(
        pl.kernel,
        out_type=out_type,
        mesh=mesh,
        compiler_params=pltpu.CompilerParams(use_tc_tiling_on_sc=False,
                                             needs_layout_passes=False),
        scratch_types=(
            shared
            + [pltpu.VMEM((C,), jnp.int32) for _ in range(2 * NPH)]
            + [pltpu.VMEM((C,), jnp.float32) for _ in range(3 * NPH)]
            + [pltpu.VMEM((C, H), jnp.float32) for _ in range(NPH)]
            + [dma for _ in range(7 * NPH)]
        ),
    )
    def sweep(*args):
        if fold:
            ei_r, ad_r, xp_r, acc_o, acc_sh = args[:5]
            as_r = den_o = den_sh = None
            scr = args[5:]
        else:
            ei_r, as_r, ad_r, xp_r, acc_o, den_o, acc_sh, den_sh = args[:8]
            scr = args[8:]
        srcv = scr[0:NPH]
        dstv = scr[NPH:2 * NPH]
        asv = scr[2 * NPH:3 * NPH]
        adv = scr[3 * NPH:4 * NPH]
        wv = scr[4 * NPH:5 * NPH]
        rowsv = scr[5 * NPH:6 * NPH]
        sems = scr[6 * NPH:]
        s_src = sems[0:NPH]
        s_dst = sems[NPH:2 * NPH]
        s_as = sems[2 * NPH:3 * NPH]
        s_ad = sems[3 * NPH:4 * NPH]
        s_rows = sems[4 * NPH:5 * NPH]
        s_sa = sems[5 * NPH:6 * NPH]
        s_sd = sems[6 * NPH:7 * NPH]

        cid = lax.axis_index("c")
        sid = lax.axis_index("s")
        wid = sid * NC + cid
        nch = jnp.where(wid < NCH_XT, NCH_LO + 1, NCH_LO)
        r0 = sid * RT

        z16 = jnp.zeros((16,), jnp.float32)
        def zfill(i, _):
            for j in range(H // 16):
                rowsv[0][i, pl.ds(j * 16, 16)] = z16
            return 0
        lax.fori_loop(0, ZC, zfill, 0)
        for j in range(ZC // 16):
            wv[0][pl.ds(j * 16, 16)] = z16
        def zfire(k, _):
            ro = r0 + k * ZC
            pltpu.async_copy(rowsv[0], acc_sh.at[pl.ds(ro, ZC)], s_sa[0])
            if not fold:
                pltpu.async_copy(wv[0], den_sh.at[pl.ds(ro, ZC)], s_sd[0])
            return 0
        lax.fori_loop(0, RT // ZC, zfire, 0)
        def zwait(k, _):
            ro = r0 + k * ZC
            pltpu.make_async_copy(rowsv[0], acc_sh.at[pl.ds(ro, ZC)],
                                  s_sa[0]).wait()
            if not fold:
                pltpu.make_async_copy(wv[0], den_sh.at[pl.ds(ro, ZC)],
                                      s_sd[0]).wait()
            return 0
        lax.fori_loop(0, RT // ZC, zwait, 0)
        plsc.subcore_barrier()

        def fire_idx(i, p):
            o = (wid + i * NW) * C
            pltpu.async_copy(ei_r.at[pl.ds(o, C)], srcv[p], s_src[p])
            pltpu.async_copy(ei_r.at[pl.ds(E + o, C)], dstv[p], s_dst[p])

        def wait_idx(p):
            pltpu.make_async_copy(ei_r.at[pl.ds(0, C)], srcv[p], s_src[p]).wait()
            pltpu.make_async_copy(ei_r.at[pl.ds(0, C)], dstv[p], s_dst[p]).wait()

        def fire_gathers(p):
            if not fold:
                pltpu.async_copy(as_r.at[srcv[p]], asv[p], s_as[p])
            pltpu.async_copy(ad_r.at[dstv[p]], adv[p], s_ad[p])
            pltpu.async_copy(xp_r.at[srcv[p]], rowsv[p], s_rows[p])

        def wait_gathers(p):
            if not fold:
                pltpu.make_async_copy(as_r.at[srcv[p]], asv[p], s_as[p]).wait()
            pltpu.make_async_copy(ad_r.at[dstv[p]], adv[p], s_ad[p]).wait()
            pltpu.make_async_copy(xp_r.at[srcv[p]], rowsv[p], s_rows[p]).wait()

        def fire_scatters(p):
            pltpu.async_copy(rowsv[p], acc_sh.at[dstv[p]], s_sa[p], add=True)
            if not fold:
                pltpu.async_copy(wv[p], den_sh.at[dstv[p]], s_sd[p], add=True)

        def wait_scatters(p):
            pltpu.make_async_copy(rowsv[p], acc_sh.at[dstv[p]], s_sa[p]).wait()
            if not fold:
                pltpu.make_async_copy(wv[p], den_sh.at[dstv[p]], s_sd[p]).wait()

        lane = jnp.arange(16, dtype=jnp.int32)
        colv = jnp.full((16,), AS_COL, jnp.int32)

        def compute(p):
            for k in range(C // 16):
                sl = pl.ds(k * 16, 16)
                if fold:
                    a_s16 = plsc.load_gather(rowsv[p], [k * 16 + lane, colv])
                else:
                    a_s16 = asv[p][sl]
                e = _leaky(a_s16 + adv[p][sl])
                wv[p][sl] = jnp.exp(e)
            def sgroup(g, _):
                w16 = wv[p][pl.ds(g * 16, 16)]
                for l in range(16):
                    i = g * 16 + l
                    w = w16[l]
                    for j in range(H // 16):
                        sl = pl.ds(j * 16, 16)
                        rowsv[p][i, sl] = rowsv[p][i, sl] * w
                return 0
            lax.fori_loop(0, C // 16, sgroup, 0)

        fire_idx(0, 0)
        fire_idx(1, 1)
        wait_idx(0)
        fire_gathers(0)

        def step(j, _):
            for l in range(NPH):
                i = j * NPH + l
                p1 = (l + 1) % NPH
                p2 = (l + 2) % NPH

                @pl.when(i + 2 < nch)
                def _():
                    @pl.when(i >= 2)
                    def _():
                        wait_scatters(p2)
                    fire_idx(i + 2, p2)

                @pl.when(i + 1 < nch)
                def _():
                    wait_idx(p1)
                    fire_gathers(p1)

                @pl.when(i < nch)
                def _():
                    wait_gathers(l)
                    compute(l)
                    fire_scatters(l)
            return 0
        lax.fori_loop(0, (NCH_LO + 1 + NPH - 1) // NPH, step, 0)
        for p in range(NPH):
            wait_scatters(p)
        plsc.subcore_barrier()

        pltpu.sync_copy(acc_sh.at[pl.ds(r0, RT)], acc_o.at[cid, pl.ds(r0, RT)])
        if not fold:
            pltpu.sync_copy(den_sh.at[pl.ds(r0, RT)],
                            den_o.at[pl.ds(cid * NP + r0, RT)])

    return sweep


_BLK = 5000
_GRID = N // _BLK


def _proj1_body(x_ref, w_ref, s_ref, d_ref, xp_ref, as_ref, ad_ref):
    xp = jnp.dot(x_ref[...], w_ref[...], preferred_element_type=jnp.float32)
    xp_ref[...] = xp
    as_ref[...] = jnp.dot(xp, s_ref[...], preferred_element_type=jnp.float32)
    ad_ref[...] = jnp.dot(xp, d_ref[...], preferred_element_type=jnp.float32)


def _proj1(x, W1t, att_s, att_d):
    return pl.pallas_call(
        _proj1_body,
        grid=(_GRID,),
        in_specs=[
            pl.BlockSpec((_BLK, IN), lambda i: (i, 0)),
            pl.BlockSpec((IN, H1), lambda i: (0, 0)),
            pl.BlockSpec((H1, 1), lambda i: (0, 0)),
            pl.BlockSpec((H1, 1), lambda i: (0, 0)),
        ],
        out_specs=[
            pl.BlockSpec((_BLK, H1), lambda i: (i, 0)),
            pl.BlockSpec((_BLK, 1), lambda i: (i, 0)),
            pl.BlockSpec((_BLK, 1), lambda i: (i, 0)),
        ],
        out_shape=[
            jax.ShapeDtypeStruct((N, H1), jnp.float32),
            jax.ShapeDtypeStruct((NP, 1), jnp.float32),
            jax.ShapeDtypeStruct((NP, 1), jnp.float32),
        ],
    )(x, W1t, att_s, att_d)


def _combine2_body(acc_ref, den_ref, b_ref, w_ref, s_ref, d_ref,
                   xg_ref, ad_ref):
    a = acc_ref[0] + acc_ref[1]
    d = den_ref[0] + den_ref[1]
    h = jnp.maximum(a / (d + 1e-16) + b_ref[...], 0.0)
    xp = jnp.dot(h, w_ref[...], preferred_element_type=jnp.float32)
    as2 = jnp.dot(xp, s_ref[...], preferred_element_type=jnp.float32)
    ad_ref[...] = jnp.dot(xp, d_ref[...], preferred_element_type=jnp.float32)
    blk = xp.shape[0]
    xg_ref[...] = jnp.concatenate(
        [xp, as2, jnp.ones((blk, 1), jnp.float32),
         jnp.zeros((blk, H1 - H2 - 2), jnp.float32)], axis=1)


def _combine_proj2(acc, den, b1, W2t, att_s, att_d):
    return pl.pallas_call(
        _combine2_body,
        grid=(_GRID,),
        in_specs=[
            pl.BlockSpec((NC, _BLK, H1), lambda i: (0, i, 0)),
            pl.BlockSpec((NC, _BLK, 1), lambda i: (0, i, 0)),
            pl.BlockSpec((1, H1), lambda i: (0, 0)),
            pl.BlockSpec((H1, H2), lambda i: (0, 0)),
            pl.BlockSpec((H2, 1), lambda i: (0, 0)),
            pl.BlockSpec((H2, 1), lambda i: (0, 0)),
        ],
        out_specs=[
            pl.BlockSpec((_BLK, H1), lambda i: (i, 0)),
            pl.BlockSpec((_BLK, 1), lambda i: (i, 0)),
        ],
        out_shape=[
            jax.ShapeDtypeStruct((N, H1), jnp.float32),
            jax.ShapeDtypeStruct((NP, 1), jnp.float32),
        ],
    )(acc, den, b1, W2t, att_s, att_d)


def _combine3_body(acc_ref, b_ref, h_ref):
    a = acc_ref[0, :, :H2] + acc_ref[1, :, :H2]
    d = acc_ref[0, :, H2 + 1:H2 + 2] + acc_ref[1, :, H2 + 1:H2 + 2]
    h_ref[...] = jnp.maximum(a / (d + 1e-16) + b_ref[...], 0.0)


def _combine3(acc, b2):
    return pl.pallas_call(
        _combine3_body,
        grid=(_GRID,),
        in_specs=[
            pl.BlockSpec((NC, _BLK, H1), lambda i: (0, i, 0)),
            pl.BlockSpec((1, H2), lambda i: (0, 0)),
        ],
        out_specs=pl.BlockSpec((_BLK, H2), lambda i: (i, 0)),
        out_shape=jax.ShapeDtypeStruct((N, H2), jnp.float32),
    )(acc, b2)


def _head_body(h_ref, w1_ref, b1_ref, w3_ref, b3_ref, ba_ref, bb_ref, o_ref):
    z = lax.dot_general(h_ref[...], w1_ref[...],
                        dimension_numbers=(((1,), (1,)), ((), ())),
                        preferred_element_type=jnp.float32)
    z = jnp.maximum(z + b1_ref[...], 0.0)
    o = jnp.dot(z, w3_ref[...], preferred_element_type=jnp.float32) + b3_ref[...]
    nbt = 1.0 + jnp.sum((ba_ref[...] != bb_ref[...]).astype(jnp.float32))
    o_ref[...] = o * (nbt / NB)


def _head(h2f, lin1_Wt, lin1_b, lin3_Wt, lin3_b, ba, bb):
    HC = h2f.shape[1]
    return pl.pallas_call(
        _head_body,
        grid=(1,),
        in_specs=[
            pl.BlockSpec((NB, HC), lambda i: (0, 0)),
            pl.BlockSpec((HC2, HC), lambda i: (0, 0)),
            pl.BlockSpec((1, HC2), lambda i: (0, 0)),
            pl.BlockSpec((HC2, 1), lambda i: (0, 0)),
            pl.BlockSpec((1, 1), lambda i: (0, 0)),
            pl.BlockSpec(ba.shape, lambda i: (0, 0)),
            pl.BlockSpec(bb.shape, lambda i: (0, 0)),
        ],
        out_specs=pl.BlockSpec((NB, 1), lambda i: (0, 0)),
        out_shape=jax.ShapeDtypeStruct((NB, 1), jnp.float32),
    )(h2f, lin1_Wt, lin1_b, lin3_Wt, lin3_b, ba, bb)


def kernel(x, edge_index, edge_weight, batch, device,
           W1, att1_src, att1_dst, b1,
           W2, att2_src, att2_dst, b2,
           lin1_W, lin1_b, lin3_W, lin3_b):
    ei_flat = edge_index.reshape(2 * E)

    xp1, as1, ad1 = _proj1(x, W1.T, att1_src.reshape(H1, 1),
                           att1_dst.reshape(H1, 1))
    acc1, den1 = _edge_sweep(H1, False)(ei_flat, as1.reshape(NP),
                                        ad1.reshape(NP), xp1)

    xg2, ad2 = _combine_proj2(acc1, den1.reshape(NC, NP, 1),
                              b1.reshape(1, H1), W2.T,
                              att2_src.reshape(H2, 1),
                              att2_dst.reshape(H2, 1))
    acc2 = _edge_sweep(H1, True)(ei_flat, ad2.reshape(NP), xg2)[0]

    h2 = _combine3(acc2, b2.reshape(1, H2))

    h2f = h2.reshape(NB, (N // NB) * H2)
    ba = batch.reshape(500, 100)
    bb = jnp.concatenate([batch[:1], batch[:-1]]).reshape(500, 100)
    out = _head(h2f, lin1_W, lin1_b.reshape(1, HC2),
                lin3_W.T, lin3_b.reshape(1, 1), ba, bb)
    return out


# speedup vs baseline: 1.7961x; 1.0003x over previous
import functools

import jax
import jax.numpy as jnp
from jax import lax
from jax.experimental import pallas as pl
from jax.experimental.pallas import tpu as pltpu
from jax.experimental.pallas import tpu_sc as plsc

N = 50000
E = 800000
IN = 115
H1 = 32
H2 = 16
NB = 50
HC2 = 64

NC = 2
NS = 16
NW = NC * NS
C = 128
NCHT = E // C
NCH_LO = NCHT // NW
NCH_XT = NCHT % NW
NP = 51200
RT = NP // NS
ZC = 128


def _leaky(v):
    return jnp.maximum(v, 0.2 * v)


def _edge_sweep(H, fold):
    mesh = plsc.VectorSubcoreMesh(core_axis_name="c", subcore_axis_name="s")

    NPH = 4
    AS_COL = H2
    dma = pltpu.SemaphoreType.DMA

    out_type = [jax.ShapeDtypeStruct((NC, NP, H), jnp.float32)]
    shared = [pltpu.VMEM_SHARED((NP, H), jnp.float32)]
    if not fold:
        out_type.append(jax.ShapeDtypeStruct((NC * NP,), jnp.float32))
        shared.append(pltpu.VMEM_SHARED((NP,), jnp.float32))

    @functools.partial(
        pl.kernel,
        out_type=out_type,
        mesh=mesh,
        compiler_params=pltpu.CompilerParams(use_tc_tiling_on_sc=False,
                                             needs_layout_passes=False),
        scratch_types=(
            shared
            + [pltpu.VMEM((C,), jnp.int32) for _ in range(2 * NPH)]
            + [pltpu.VMEM((C,), jnp.float32) for _ in range(3 * NPH)]
            + [pltpu.VMEM((C, H), jnp.float32) for _ in range(NPH)]
            + [dma for _ in range(7 * NPH)]
        ),
    )
    def sweep(*args):
        if fold:
            ei_r, ad_r, xp_r, acc_o, acc_sh = args[:5]
            as_r = den_o = den_sh = None
            scr = args[5:]
        else:
            ei_r, as_r, ad_r, xp_r, acc_o, den_o, acc_sh, den_sh = args[:8]
            scr = args[8:]
        srcv = scr[0:NPH]
        dstv = scr[NPH:2 * NPH]
        asv = scr[2 * NPH:3 * NPH]
        adv = scr[3 * NPH:4 * NPH]
        wv = scr[4 * NPH:5 * NPH]
        rowsv = scr[5 * NPH:6 * NPH]
        sems = scr[6 * NPH:]
        s_src = sems[0:NPH]
        s_dst = sems[NPH:2 * NPH]
        s_as = sems[2 * NPH:3 * NPH]
        s_ad = sems[3 * NPH:4 * NPH]
        s_rows = sems[4 * NPH:5 * NPH]
        s_sa = sems[5 * NPH:6 * NPH]
        s_sd = sems[6 * NPH:7 * NPH]

        cid = lax.axis_index("c")
        sid = lax.axis_index("s")
        wid = sid * NC + cid
        nch = jnp.where(wid < NCH_XT, NCH_LO + 1, NCH_LO)
        r0 = sid * RT

        z16 = jnp.zeros((16,), jnp.float32)
        def zfill(i, _):
            for j in range(H // 16):
                rowsv[0][i, pl.ds(j * 16, 16)] = z16
            return 0
        lax.fori_loop(0, ZC, zfill, 0)
        for j in range(ZC // 16):
            wv[0][pl.ds(j * 16, 16)] = z16
        def zfire(k, _):
            ro = r0 + k * ZC
            pltpu.async_copy(rowsv[0], acc_sh.at[pl.ds(ro, ZC)], s_sa[0])
            if not fold:
                pltpu.async_copy(wv[0], den_sh.at[pl.ds(ro, ZC)], s_sd[0])
            return 0
        lax.fori_loop(0, RT // ZC, zfire, 0)
        def zwait(k, _):
            ro = r0 + k * ZC
            pltpu.make_async_copy(rowsv[0], acc_sh.at[pl.ds(ro, ZC)],
                                  s_sa[0]).wait()
            if not fold:
                pltpu.make_async_copy(wv[0], den_sh.at[pl.ds(ro, ZC)],
                                      s_sd[0]).wait()
            return 0
        lax.fori_loop(0, RT // ZC, zwait, 0)
        plsc.subcore_barrier()

        def fire_idx(i, p):
            o = (wid + i * NW) * C
            pltpu.async_copy(ei_r.at[pl.ds(o, C)], srcv[p], s_src[p])
            pltpu.async_copy(ei_r.at[pl.ds(E + o, C)], dstv[p], s_dst[p])

        def wait_idx(p):
            pltpu.make_async_copy(ei_r.at[pl.ds(0, C)], srcv[p], s_src[p]).wait()
            pltpu.make_async_copy(ei_r.at[pl.ds(0, C)], dstv[p], s_dst[p]).wait()

        def fire_gathers(p):
            if not fold:
                pltpu.async_copy(as_r.at[srcv[p]], asv[p], s_as[p])
            pltpu.async_copy(ad_r.at[dstv[p]], adv[p], s_ad[p])
            pltpu.async_copy(xp_r.at[srcv[p]], rowsv[p], s_rows[p])

        def wait_gathers(p):
            if not fold:
                pltpu.make_async_copy(as_r.at[srcv[p]], asv[p], s_as[p]).wait()
            pltpu.make_async_copy(ad_r.at[dstv[p]], adv[p], s_ad[p]).wait()
            pltpu.make_async_copy(xp_r.at[srcv[p]], rowsv[p], s_rows[p]).wait()

        def fire_scatters(p):
            pltpu.async_copy(rowsv[p], acc_sh.at[dstv[p]], s_sa[p], add=True)
            if not fold:
                pltpu.async_copy(wv[p], den_sh.at[dstv[p]], s_sd[p], add=True)

        def wait_scatters(p):
            pltpu.make_async_copy(rowsv[p], acc_sh.at[dstv[p]], s_sa[p]).wait()
            if not fold:
                pltpu.make_async_copy(wv[p], den_sh.at[dstv[p]], s_sd[p]).wait()

        lane = jnp.arange(16, dtype=jnp.int32)
        colv = jnp.full((16,), AS_COL, jnp.int32)

        def compute(p):
            for k in range(C // 16):
                sl = pl.ds(k * 16, 16)
                if fold:
                    a_s16 = plsc.load_gather(rowsv[p], [k * 16 + lane, colv])
                else:
                    a_s16 = asv[p][sl]
                e = _leaky(a_s16 + adv[p][sl])
                wv[p][sl] = jnp.exp(e)
            def sgroup(g, _):
                w16 = wv[p][pl.ds(g * 16, 16)]
                for l in range(16):
                    i = g * 16 + l
                    w = w16[l]
                    for j in range(H // 16):
                        sl = pl.ds(j * 16, 16)
                        rowsv[p][i, sl] = rowsv[p][i, sl] * w
                return 0
            lax.fori_loop(0, C // 16, sgroup, 0)

        fire_idx(0, 0)
        fire_idx(1, 1)
        wait_idx(0)
        fire_gathers(0)

        def step(j, _):
            for l in range(NPH):
                i = j * NPH + l
                p1 = (l + 1) % NPH
                p2 = (l + 2) % NPH

                @pl.when(i + 2 < nch)
                def _():
                    @pl.when(i >= 2)
                    def _():
                        wait_scatters(p2)
                    fire_idx(i + 2, p2)

                @pl.when(i + 1 < nch)
                def _():
                    wait_idx(p1)
                    fire_gathers(p1)

                @pl.when(i < nch)
                def _():
                    wait_gathers(l)
                    compute(l)
                    fire_scatters(l)
            return 0
        lax.fori_loop(0, (NCH_LO + 1 + NPH - 1) // NPH, step, 0)
        for p in range(NPH):
            wait_scatters(p)
        plsc.subcore_barrier()

        pltpu.sync_copy(acc_sh.at[pl.ds(r0, RT)], acc_o.at[cid, pl.ds(r0, RT)])
        if not fold:
            pltpu.sync_copy(den_sh.at[pl.ds(r0, RT)],
                            den_o.at[pl.ds(cid * NP + r0, RT)])

    return sweep


_BLK = 5000
_GRID = N // _BLK


def _proj1_body(x_ref, w_ref, s_ref, d_ref, xp_ref, as_ref, ad_ref):
    xp = jnp.dot(x_ref[...], w_ref[...], preferred_element_type=jnp.float32)
    xp_ref[...] = xp
    as_ref[...] = jnp.dot(xp, s_ref[...], preferred_element_type=jnp.float32)
    ad_ref[...] = jnp.dot(xp, d_ref[...], preferred_element_type=jnp.float32)


def _proj1(x, W1t, att_s, att_d):
    return pl.pallas_call(
        _proj1_body,
        grid=(_GRID,),
        in_specs=[
            pl.BlockSpec((_BLK, IN), lambda i: (i, 0)),
            pl.BlockSpec((IN, H1), lambda i: (0, 0)),
            pl.BlockSpec((H1, 1), lambda i: (0, 0)),
            pl.BlockSpec((H1, 1), lambda i: (0, 0)),
        ],
        out_specs=[
            pl.BlockSpec((_BLK, H1), lambda i: (i, 0)),
            pl.BlockSpec((_BLK, 1), lambda i: (i, 0)),
            pl.BlockSpec((_BLK, 1), lambda i: (i, 0)),
        ],
        out_shape=[
            jax.ShapeDtypeStruct((N, H1), jnp.float32),
            jax.ShapeDtypeStruct((NP, 1), jnp.float32),
            jax.ShapeDtypeStruct((NP, 1), jnp.float32),
        ],
    )(x, W1t, att_s, att_d)


def _combine2_body(acc_ref, den_ref, b_ref, w_ref, s_ref, d_ref,
                   xg_ref, ad_ref):
    a = acc_ref[0] + acc_ref[1]
    d = den_ref[0] + den_ref[1]
    h = jnp.maximum(a / (d + 1e-16) + b_ref[...], 0.0)
    xp = jnp.dot(h, w_ref[...], preferred_element_type=jnp.float32)
    as2 = jnp.dot(xp, s_ref[...], preferred_element_type=jnp.float32)
    ad_ref[...] = jnp.dot(xp, d_ref[...], preferred_element_type=jnp.float32)
    blk = xp.shape[0]
    xg_ref[...] = jnp.concatenate(
        [xp, as2, jnp.ones((blk, 1), jnp.float32),
         jnp.zeros((blk, H1 - H2 - 2), jnp.float32)], axis=1)


def _combine_proj2(acc, den, b1, W2t, att_s, att_d):
    return pl.pallas_call(
        _combine2_body,
        grid=(_GRID,),
        in_specs=[
            pl.BlockSpec((NC, _BLK, H1), lambda i: (0, i, 0)),
            pl.BlockSpec((NC, _BLK, 1), lambda i: (0, i, 0)),
            pl.BlockSpec((1, H1), lambda i: (0, 0)),
            pl.BlockSpec((H1, H2), lambda i: (0, 0)),
            pl.BlockSpec((H2, 1), lambda i: (0, 0)),
            pl.BlockSpec((H2, 1), lambda i: (0, 0)),
        ],
        out_specs=[
            pl.BlockSpec((_BLK, H1), lambda i: (i, 0)),
            pl.BlockSpec((_BLK, 1), lambda i: (i, 0)),
        ],
        out_shape=[
            jax.ShapeDtypeStruct((N, H1), jnp.float32),
            jax.ShapeDtypeStruct((NP, 1), jnp.float32),
        ],
    )(acc, den, b1, W2t, att_s, att_d)


def _combine3_body(acc_ref, b_ref, h_ref):
    a = acc_ref[0, :, :H2] + acc_ref[1, :, :H2]
    d = acc_ref[0, :, H2 + 1:H2 + 2] + acc_ref[1, :, H2 + 1:H2 + 2]
    h_ref[...] = jnp.maximum(a / (d + 1e-16) + b_ref[...], 0.0)


def _combine3(acc, b2):
    return pl.pallas_call(
        _combine3_body,
        grid=(_GRID,),
        in_specs=[
            pl.BlockSpec((NC, _BLK, H1), lambda i: (0, i, 0)),
            pl.BlockSpec((1, H2), lambda i: (0, 0)),
        ],
        out_specs=pl.BlockSpec((_BLK, H2), lambda i: (i, 0)),
        out_shape=jax.ShapeDtypeStruct((N, H2), jnp.float32),
    )(acc, b2)


def _head_body(h_ref, w1_ref, b1_ref, w3_ref, b3_ref, ba_ref, o_ref):
    z = lax.dot_general(h_ref[...], w1_ref[...],
                        dimension_numbers=(((1,), (1,)), ((), ())),
                        preferred_element_type=jnp.float32)
    z = jnp.maximum(z + b1_ref[...], 0.0)
    o = jnp.dot(z, w3_ref[...], preferred_element_type=jnp.float32) + b3_ref[...]
    ba = ba_ref[...]
    nbt = (1.0
           + jnp.sum((ba[:, 1:] != ba[:, :-1]).astype(jnp.float32))
           + jnp.sum((ba[1:, :1] != ba[:-1, -1:]).astype(jnp.float32)))
    o_ref[...] = o * (nbt / NB)


def _head(h2f, lin1_Wt, lin1_b, lin3_Wt, lin3_b, ba):
    HC = h2f.shape[1]
    return pl.pallas_call(
        _head_body,
        grid=(1,),
        in_specs=[
            pl.BlockSpec((NB, HC), lambda i: (0, 0)),
            pl.BlockSpec((HC2, HC), lambda i: (0, 0)),
            pl.BlockSpec((1, HC2), lambda i: (0, 0)),
            pl.BlockSpec((HC2, 1), lambda i: (0, 0)),
            pl.BlockSpec((1, 1), lambda i: (0, 0)),
            pl.BlockSpec(ba.shape, lambda i: (0, 0)),
        ],
        out_specs=pl.BlockSpec((NB, 1), lambda i: (0, 0)),
        out_shape=jax.ShapeDtypeStruct((NB, 1), jnp.float32),
    )(h2f, lin1_Wt, lin1_b, lin3_Wt, lin3_b, ba)


def kernel(x, edge_index, edge_weight, batch, device,
           W1, att1_src, att1_dst, b1,
           W2, att2_src, att2_dst, b2,
           lin1_W, lin1_b, lin3_W, lin3_b):
    ei_flat = edge_index.reshape(2 * E)

    xp1, as1, ad1 = _proj1(x, W1.T, att1_src.reshape(H1, 1),
                           att1_dst.reshape(H1, 1))
    acc1, den1 = _edge_sweep(H1, False)(ei_flat, as1.reshape(NP),
                                        ad1.reshape(NP), xp1)

    xg2, ad2 = _combine_proj2(acc1, den1.reshape(NC, NP, 1),
                              b1.reshape(1, H1), W2.T,
                              att2_src.reshape(H2, 1),
                              att2_dst.reshape(H2, 1))
    acc2 = _edge_sweep(H1, True)(ei_flat, ad2.reshape(NP), xg2)[0]

    h2 = _combine3(acc2, b2.reshape(1, H2))

    h2f = h2.reshape(NB, (N // NB) * H2)
    ba = batch.reshape(500, 100)
    out = _head(h2f, lin1_W, lin1_b.reshape(1, HC2),
                lin3_W.T, lin3_b.reshape(1, 1), ba)
    return out
